# baseline (speedup 1.0000x reference)
import math

import jax
import jax.numpy as jnp
from jax.experimental import pallas as pl
from jax.experimental.pallas import tpu as pltpu

_Q = 2048
_KV = 2048
_D = 1024
_B = 2
_C = 5
_U_ACT = int(_C * math.log(_Q))
_U_SAMP = int(_Q * math.log(_KV))
_CAND = 64
_PAD = 64
_QCHUNK = 512
_NQ = _Q // _QCHUNK
_OCHUNK = 256
_NO = _Q // _OCHUNK
_NEG_INF = float("-inf")


def _m_kernel(w_ref, bias_ref, q_ref, k_ref, m_ref):
    s = jax.lax.dot_general(q_ref[0], k_ref[0], (((1,), (1,)), ((), ())),
                            preferred_element_type=jnp.float32)
    mx = jnp.max(s + bias_ref[0], axis=1)
    mean = jnp.sum(s * w_ref[0], axis=1)
    m_ref[0, 0, :] = mx - mean


def _select_attend_kernel(m_ref, w_ref, bias_ref, q_ref, k_ref, v_ref,
                          s1_ref, idx_ref, vmean_ref):
    M2 = m_ref[0, 0, :].reshape(16, 128)
    flat = (jax.lax.broadcasted_iota(jnp.int32, (16, 128), 0) * 128
            + jax.lax.broadcasted_iota(jnp.int32, (16, 128), 1))
    slot_col = jax.lax.broadcasted_iota(jnp.int32, (_CAND, 1), 0)
    slot_row = jax.lax.broadcasted_iota(jnp.int32, (1, _CAND), 1)

    def cand_body(i, carry):
        m2, idx_col, idx_row = carry
        mx = jnp.max(m2)
        idx = jnp.min(jnp.where(m2 == mx, flat, jnp.int32(_Q)))
        idx_col = jnp.where(slot_col == i, idx, idx_col)
        idx_row = jnp.where(slot_row == i, idx, idx_row)
        m2 = jnp.where(flat == idx, _NEG_INF, m2)
        return m2, idx_col, idx_row

    idx_col0 = jnp.full((_CAND, 1), -1, jnp.int32)
    idx_row0 = jnp.full((1, _CAND), -1, jnp.int32)
    _, cand_col, cand_row = jax.lax.fori_loop(
        0, _CAND, cand_body, (M2, idx_col0, idx_row0))

    onehot = (jax.lax.broadcasted_iota(jnp.int32, (_CAND, _Q), 1)
              == cand_col).astype(jnp.float32)
    q_cand = jax.lax.dot_general(onehot, q_ref[0], (((1,), (0,)), ((), ())),
                                 preferred_element_type=jnp.float32)
    s_cand = jax.lax.dot_general(q_cand, k_ref[0], (((1,), (1,)), ((), ())),
                                 preferred_element_type=jnp.float32)
    m_cand = (jnp.max(s_cand + bias_ref[0], axis=1, keepdims=True)
              - jnp.sum(s_cand * w_ref[0], axis=1, keepdims=True))

    out_row = jax.lax.broadcasted_iota(jnp.int32, (_PAD, _CAND), 0)

    def sel_body(j, carry):
        mc, lmat, fin_row = carry
        mx = jnp.max(mc)
        orig = jnp.min(jnp.where(mc == mx, cand_col, jnp.int32(_Q)))
        slot_mask_col = cand_col == orig
        slot_mask_row = cand_row == orig
        lmat = jnp.where((out_row == j) & slot_mask_row, 1.0, lmat)
        fin_row = jnp.where(slot_row == j, orig, fin_row)
        mc = jnp.where(slot_mask_col, _NEG_INF, mc)
        return mc, lmat, fin_row

    lmat0 = jnp.zeros((_PAD, _CAND), jnp.float32)
    _, lmat, fin_row = jax.lax.fori_loop(
        0, _U_ACT, sel_body, (m_cand, lmat0, idx_row0))
    idx_ref[0, 0, :] = fin_row[0, :]

    scale = 1.0 / math.sqrt(_KV)
    att = jax.lax.dot_general(lmat, s_cand, (((1,), (0,)), ((), ())),
                              preferred_element_type=jnp.float32) * scale
    att = att - jnp.max(att, axis=1, keepdims=True)
    att = jnp.exp(att)
    att = att / jnp.sum(att, axis=1, keepdims=True)
    s1_ref[0] = jax.lax.dot_general(att, v_ref[0], (((1,), (0,)), ((), ())),
                                    preferred_element_type=jnp.float32)
    vmean_ref[0, 0, :] = jnp.mean(v_ref[0], axis=0)


def _output_kernel(s1_ref, idx_ref, vmean_ref, out_ref):
    c = pl.program_id(1)
    onehot_t = ((jax.lax.broadcasted_iota(jnp.int32, (_OCHUNK, _PAD), 0)
                 + c * _OCHUNK) == idx_ref[0]).astype(jnp.float32)
    scattered = jax.lax.dot_general(
        onehot_t, s1_ref[0], (((1,), (0,)), ((), ())),
        preferred_element_type=jnp.float32)
    unsel = 1.0 - jnp.sum(onehot_t, axis=1, keepdims=True)
    out_ref[0] = scattered + unsel * vmean_ref[0]


def kernel(q, k, v):
    idx = jax.random.randint(jax.random.key(42), (_B, _U_SAMP), 0, _KV)
    counts = jax.vmap(
        lambda ix: jnp.zeros((_KV,), jnp.float32).at[ix].add(1.0))(idx)
    w = (counts / _U_SAMP).reshape(_B, 1, _KV)
    bias = jnp.where(counts > 0, 0.0, _NEG_INF).astype(jnp.float32)
    bias = bias.reshape(_B, 1, _KV)
    q16 = q.astype(jnp.bfloat16)
    k16 = k.astype(jnp.bfloat16)

    M = pl.pallas_call(
        _m_kernel,
        grid=(_B, _NQ),
        in_specs=[
            pl.BlockSpec((1, 1, _KV), lambda b, i: (b, 0, 0)),
            pl.BlockSpec((1, 1, _KV), lambda b, i: (b, 0, 0)),
            pl.BlockSpec((1, _QCHUNK, _D), lambda b, i: (b, i, 0)),
            pl.BlockSpec((1, _KV, _D), lambda b, i: (b, 0, 0)),
        ],
        out_specs=pl.BlockSpec((1, 1, _QCHUNK), lambda b, i: (b, 0, i)),
        out_shape=jax.ShapeDtypeStruct((_B, 1, _Q), jnp.float32),
        compiler_params=pltpu.CompilerParams(
            dimension_semantics=("arbitrary", "arbitrary")),
    )(w, bias, q16, k16)

    s1, top_idx, v_mean = pl.pallas_call(
        _select_attend_kernel,
        grid=(_B,),
        in_specs=[
            pl.BlockSpec((1, 1, _Q), lambda b: (b, 0, 0)),
            pl.BlockSpec((1, 1, _KV), lambda b: (b, 0, 0)),
            pl.BlockSpec((1, 1, _KV), lambda b: (b, 0, 0)),
            pl.BlockSpec((1, _Q, _D), lambda b: (b, 0, 0)),
            pl.BlockSpec((1, _KV, _D), lambda b: (b, 0, 0)),
            pl.BlockSpec((1, _KV, _D), lambda b: (b, 0, 0)),
        ],
        out_specs=[
            pl.BlockSpec((1, _PAD, _D), lambda b: (b, 0, 0)),
            pl.BlockSpec((1, 1, _PAD), lambda b: (b, 0, 0)),
            pl.BlockSpec((1, 1, _D), lambda b: (b, 0, 0)),
        ],
        out_shape=[
            jax.ShapeDtypeStruct((_B, _PAD, _D), jnp.float32),
            jax.ShapeDtypeStruct((_B, 1, _PAD), jnp.int32),
            jax.ShapeDtypeStruct((_B, 1, _D), jnp.float32),
        ],
        compiler_params=pltpu.CompilerParams(
            dimension_semantics=("arbitrary",)),
    )(M, w, bias, q, k, v)

    return pl.pallas_call(
        _output_kernel,
        grid=(_B, _NO),
        in_specs=[
            pl.BlockSpec((1, _PAD, _D), lambda b, i: (b, 0, 0)),
            pl.BlockSpec((1, 1, _PAD), lambda b, i: (b, 0, 0)),
            pl.BlockSpec((1, 1, _D), lambda b, i: (b, 0, 0)),
        ],
        out_specs=pl.BlockSpec((1, _OCHUNK, _D), lambda b, i: (b, i, 0)),
        out_shape=jax.ShapeDtypeStruct((_B, _Q, _D), jnp.float32),
        compiler_params=pltpu.CompilerParams(
            dimension_semantics=("arbitrary", "arbitrary")),
    )(s1, top_idx, v_mean)


# speedup vs baseline: 2.7578x; 2.7578x over previous
import math

import jax
import jax.numpy as jnp
import numpy as np
from jax.experimental import pallas as pl
from jax.experimental.pallas import tpu as pltpu

_Q = 2048
_KV = 2048
_D = 1024
_B = 2
_C = 5
_U_ACT = int(_C * math.log(_Q))
_U_SAMP = int(_Q * math.log(_KV))
_CAND = 64
_PAD = 64
_QCHUNK = 512
_NQ = _Q // _QCHUNK
_OCHUNK = 256
_NO = _Q // _OCHUNK
_NEG_INF = float("-inf")


def _m_kernel(w_ref, bias_ref, q_ref, k_ref, m_ref):
    qb = q_ref[0].astype(jnp.bfloat16)
    kb = k_ref[0].astype(jnp.bfloat16)
    s = jax.lax.dot_general(qb, kb, (((1,), (1,)), ((), ())),
                            preferred_element_type=jnp.float32)
    mx = jnp.max(s + bias_ref[0], axis=1)
    mean = jnp.sum(s * w_ref[0], axis=1)
    m_ref[0, 0, :] = mx - mean


def _select_attend_kernel(m_ref, w_ref, bias_ref, q_ref, k_ref, v_ref,
                          s1_ref, idx_ref, vmean_ref):
    M2 = m_ref[0, 0, :].reshape(16, 128)
    flat = (jax.lax.broadcasted_iota(jnp.int32, (16, 128), 0) * 128
            + jax.lax.broadcasted_iota(jnp.int32, (16, 128), 1))
    slot_col = jax.lax.broadcasted_iota(jnp.int32, (_CAND, 1), 0)
    slot_row = jax.lax.broadcasted_iota(jnp.int32, (1, _CAND), 1)

    def cand_body(i, carry):
        m2, idx_col, idx_row = carry
        mx = jnp.max(m2)
        idx = jnp.min(jnp.where(m2 == mx, flat, jnp.int32(_Q)))
        idx_col = jnp.where(slot_col == i, idx, idx_col)
        idx_row = jnp.where(slot_row == i, idx, idx_row)
        m2 = jnp.where(flat == idx, _NEG_INF, m2)
        return m2, idx_col, idx_row

    idx_col0 = jnp.full((_CAND, 1), -1, jnp.int32)
    idx_row0 = jnp.full((1, _CAND), -1, jnp.int32)
    _, cand_col, cand_row = jax.lax.fori_loop(
        0, _CAND, cand_body, (M2, idx_col0, idx_row0))

    onehot = (jax.lax.broadcasted_iota(jnp.int32, (_CAND, _Q), 1)
              == cand_col).astype(jnp.float32)
    q_cand = jax.lax.dot_general(onehot, q_ref[0], (((1,), (0,)), ((), ())),
                                 preferred_element_type=jnp.float32)
    s_cand = jax.lax.dot_general(q_cand, k_ref[0], (((1,), (1,)), ((), ())),
                                 preferred_element_type=jnp.float32)
    m_cand = (jnp.max(s_cand + bias_ref[0], axis=1, keepdims=True)
              - jnp.sum(s_cand * w_ref[0], axis=1, keepdims=True))

    out_row = jax.lax.broadcasted_iota(jnp.int32, (_PAD, _CAND), 0)

    def sel_body(j, carry):
        mc, lmat, fin_row = carry
        mx = jnp.max(mc)
        orig = jnp.min(jnp.where(mc == mx, cand_col, jnp.int32(_Q)))
        slot_mask_col = cand_col == orig
        slot_mask_row = cand_row == orig
        lmat = jnp.where((out_row == j) & slot_mask_row, 1.0, lmat)
        fin_row = jnp.where(slot_row == j, orig, fin_row)
        mc = jnp.where(slot_mask_col, _NEG_INF, mc)
        return mc, lmat, fin_row

    lmat0 = jnp.zeros((_PAD, _CAND), jnp.float32)
    _, lmat, fin_row = jax.lax.fori_loop(
        0, _U_ACT, sel_body, (m_cand, lmat0, idx_row0))
    idx_ref[0, 0, :] = fin_row[0, :]

    scale = 1.0 / math.sqrt(_KV)
    att = jax.lax.dot_general(lmat, s_cand, (((1,), (0,)), ((), ())),
                              preferred_element_type=jnp.float32) * scale
    att = att - jnp.max(att, axis=1, keepdims=True)
    att = jnp.exp(att)
    att = att / jnp.sum(att, axis=1, keepdims=True)
    s1_ref[0] = jax.lax.dot_general(att, v_ref[0], (((1,), (0,)), ((), ())),
                                    preferred_element_type=jnp.float32)
    vmean_ref[0, 0, :] = jnp.mean(v_ref[0], axis=0)


def _output_kernel(s1_ref, idx_ref, vmean_ref, out_ref):
    c = pl.program_id(1)
    onehot_t = ((jax.lax.broadcasted_iota(jnp.int32, (_OCHUNK, _PAD), 0)
                 + c * _OCHUNK) == idx_ref[0]).astype(jnp.float32)
    scattered = jax.lax.dot_general(
        onehot_t, s1_ref[0], (((1,), (0,)), ((), ())),
        preferred_element_type=jnp.float32)
    unsel = 1.0 - jnp.sum(onehot_t, axis=1, keepdims=True)
    out_ref[0] = scattered + unsel * vmean_ref[0]


_IDX = np.asarray(jax.random.randint(jax.random.key(42), (_B, _U_SAMP),
                                     0, _KV))
_COUNTS = np.stack([np.bincount(_IDX[b], minlength=_KV)
                    for b in range(_B)]).astype(np.float32)
_W_HOST = (_COUNTS / np.float32(_U_SAMP)).reshape(_B, 1, _KV)
_BIAS_HOST = np.where(_COUNTS > 0, np.float32(0.0),
                      np.float32(_NEG_INF)).reshape(_B, 1, _KV)


def kernel(q, k, v):
    w = jnp.asarray(_W_HOST)
    bias = jnp.asarray(_BIAS_HOST)

    M = pl.pallas_call(
        _m_kernel,
        grid=(_B, _NQ),
        in_specs=[
            pl.BlockSpec((1, 1, _KV), lambda b, i: (b, 0, 0)),
            pl.BlockSpec((1, 1, _KV), lambda b, i: (b, 0, 0)),
            pl.BlockSpec((1, _QCHUNK, _D), lambda b, i: (b, i, 0)),
            pl.BlockSpec((1, _KV, _D), lambda b, i: (b, 0, 0)),
        ],
        out_specs=pl.BlockSpec((1, 1, _QCHUNK), lambda b, i: (b, 0, i)),
        out_shape=jax.ShapeDtypeStruct((_B, 1, _Q), jnp.float32),
        compiler_params=pltpu.CompilerParams(
            dimension_semantics=("arbitrary", "arbitrary")),
    )(w, bias, q, k)

    s1, top_idx, v_mean = pl.pallas_call(
        _select_attend_kernel,
        grid=(_B,),
        in_specs=[
            pl.BlockSpec((1, 1, _Q), lambda b: (b, 0, 0)),
            pl.BlockSpec((1, 1, _KV), lambda b: (b, 0, 0)),
            pl.BlockSpec((1, 1, _KV), lambda b: (b, 0, 0)),
            pl.BlockSpec((1, _Q, _D), lambda b: (b, 0, 0)),
            pl.BlockSpec((1, _KV, _D), lambda b: (b, 0, 0)),
            pl.BlockSpec((1, _KV, _D), lambda b: (b, 0, 0)),
        ],
        out_specs=[
            pl.BlockSpec((1, _PAD, _D), lambda b: (b, 0, 0)),
            pl.BlockSpec((1, 1, _PAD), lambda b: (b, 0, 0)),
            pl.BlockSpec((1, 1, _D), lambda b: (b, 0, 0)),
        ],
        out_shape=[
            jax.ShapeDtypeStruct((_B, _PAD, _D), jnp.float32),
            jax.ShapeDtypeStruct((_B, 1, _PAD), jnp.int32),
            jax.ShapeDtypeStruct((_B, 1, _D), jnp.float32),
        ],
        compiler_params=pltpu.CompilerParams(
            dimension_semantics=("arbitrary",)),
    )(M, w, bias, q, k, v)

    return pl.pallas_call(
        _output_kernel,
        grid=(_B, _NO),
        in_specs=[
            pl.BlockSpec((1, _PAD, _D), lambda b, i: (b, 0, 0)),
            pl.BlockSpec((1, 1, _PAD), lambda b, i: (b, 0, 0)),
            pl.BlockSpec((1, 1, _D), lambda b, i: (b, 0, 0)),
        ],
        out_specs=pl.BlockSpec((1, _OCHUNK, _D), lambda b, i: (b, i, 0)),
        out_shape=jax.ShapeDtypeStruct((_B, _Q, _D), jnp.float32),
        compiler_params=pltpu.CompilerParams(
            dimension_semantics=("arbitrary", "arbitrary")),
    )(s1, top_idx, v_mean)


# speedup vs baseline: 2.7584x; 1.0003x over previous
import math

import jax
import jax.numpy as jnp
import numpy as np
from jax.experimental import pallas as pl
from jax.experimental.pallas import tpu as pltpu

_Q = 2048
_KV = 2048
_D = 1024
_B = 2
_C = 5
_U_ACT = int(_C * math.log(_Q))
_U_SAMP = int(_Q * math.log(_KV))
_CAND = 64
_PAD = 64
_QCHUNK = 512
_NQ = _Q // _QCHUNK
_OCHUNK = 256
_NO = _Q // _OCHUNK
_NEG_INF = float("-inf")


def _m_kernel(w_ref, bias_ref, q_ref, k_ref, m_ref):
    qb = q_ref[0].astype(jnp.bfloat16)
    kb = k_ref[0].astype(jnp.bfloat16)
    s = jax.lax.dot_general(qb, kb, (((1,), (1,)), ((), ())),
                            preferred_element_type=jnp.float32)
    mx = jnp.max(s + bias_ref[0], axis=1)
    mean = jnp.sum(s * w_ref[0], axis=1)
    m_ref[0, 0, :] = mx - mean


def _select_attend_kernel(m_ref, w_ref, bias_ref, q_ref, k_ref, v_ref,
                          s1_ref, idx_ref, vmean_ref):
    M2 = m_ref[0, 0, :].reshape(16, 128)
    flat = (jax.lax.broadcasted_iota(jnp.int32, (16, 128), 0) * 128
            + jax.lax.broadcasted_iota(jnp.int32, (16, 128), 1))
    slot_col = jax.lax.broadcasted_iota(jnp.int32, (_CAND, 1), 0)
    slot_row = jax.lax.broadcasted_iota(jnp.int32, (1, _CAND), 1)

    def cand_body(i, carry):
        m2, idx_col, idx_row = carry
        mx = jnp.max(m2)
        idx = jnp.min(jnp.where(m2 == mx, flat, jnp.int32(_Q)))
        idx_col = jnp.where(slot_col == i, idx, idx_col)
        idx_row = jnp.where(slot_row == i, idx, idx_row)
        m2 = jnp.where(flat == idx, _NEG_INF, m2)
        return m2, idx_col, idx_row

    idx_col0 = jnp.full((_CAND, 1), -1, jnp.int32)
    idx_row0 = jnp.full((1, _CAND), -1, jnp.int32)
    _, cand_col, cand_row = jax.lax.fori_loop(
        0, _CAND, cand_body, (M2, idx_col0, idx_row0))

    onehot = (jax.lax.broadcasted_iota(jnp.int32, (_CAND, _Q), 1)
              == cand_col).astype(jnp.float32)
    q_cand = jax.lax.dot_general(onehot, q_ref[0], (((1,), (0,)), ((), ())),
                                 preferred_element_type=jnp.float32)
    s_cand = jax.lax.dot_general(q_cand, k_ref[0], (((1,), (1,)), ((), ())),
                                 preferred_element_type=jnp.float32)
    m_cand = (jnp.max(s_cand + bias_ref[0], axis=1, keepdims=True)
              - jnp.sum(s_cand * w_ref[0], axis=1, keepdims=True))

    out_row = jax.lax.broadcasted_iota(jnp.int32, (_PAD, _CAND), 0)

    def sel_body(j, carry):
        mc, lmat, fin_row = carry
        mx = jnp.max(mc)
        orig = jnp.min(jnp.where(mc == mx, cand_col, jnp.int32(_Q)))
        slot_mask_col = cand_col == orig
        slot_mask_row = cand_row == orig
        lmat = jnp.where((out_row == j) & slot_mask_row, 1.0, lmat)
        fin_row = jnp.where(slot_row == j, orig, fin_row)
        mc = jnp.where(slot_mask_col, _NEG_INF, mc)
        return mc, lmat, fin_row

    lmat0 = jnp.zeros((_PAD, _CAND), jnp.float32)
    _, lmat, fin_row = jax.lax.fori_loop(
        0, _U_ACT, sel_body, (m_cand, lmat0, idx_row0))
    idx_ref[0, 0, :] = fin_row[0, :]

    scale = 1.0 / math.sqrt(_KV)
    att = jax.lax.dot_general(lmat, s_cand, (((1,), (0,)), ((), ())),
                              preferred_element_type=jnp.float32) * scale
    att = att - jnp.max(att, axis=1, keepdims=True)
    att = jnp.exp(att)
    att = att / jnp.sum(att, axis=1, keepdims=True)
    s1_ref[0] = jax.lax.dot_general(att, v_ref[0], (((1,), (0,)), ((), ())),
                                    preferred_element_type=jnp.float32)
    vmean_ref[0, 0, :] = jnp.mean(v_ref[0], axis=0)


def _output_kernel(s1_ref, idx_ref, vmean_ref, out_ref):
    c = pl.program_id(1)
    onehot_t = ((jax.lax.broadcasted_iota(jnp.int32, (_OCHUNK, _PAD), 0)
                 + c * _OCHUNK) == idx_ref[0]).astype(jnp.float32)
    scattered = jax.lax.dot_general(
        onehot_t, s1_ref[0], (((1,), (0,)), ((), ())),
        preferred_element_type=jnp.float32)
    unsel = 1.0 - jnp.sum(onehot_t, axis=1, keepdims=True)
    out_ref[0] = scattered + unsel * vmean_ref[0]


_IDX = np.asarray(jax.random.randint(jax.random.key(42), (_B, _U_SAMP),
                                     0, _KV))
_COUNTS = np.stack([np.bincount(_IDX[b], minlength=_KV)
                    for b in range(_B)]).astype(np.float32)
_W_HOST = (_COUNTS / np.float32(_U_SAMP)).reshape(_B, 1, _KV)
_BIAS_HOST = np.where(_COUNTS > 0, np.float32(0.0),
                      np.float32(_NEG_INF)).reshape(_B, 1, _KV)


def kernel(q, k, v):
    w = jnp.asarray(_W_HOST)
    bias = jnp.asarray(_BIAS_HOST)

    M = pl.pallas_call(
        _m_kernel,
        grid=(_B, _NQ),
        in_specs=[
            pl.BlockSpec((1, 1, _KV), lambda b, i: (b, 0, 0)),
            pl.BlockSpec((1, 1, _KV), lambda b, i: (b, 0, 0)),
            pl.BlockSpec((1, _QCHUNK, _D), lambda b, i: (b, i, 0)),
            pl.BlockSpec((1, _KV, _D), lambda b, i: (b, 0, 0)),
        ],
        out_specs=pl.BlockSpec((1, 1, _QCHUNK), lambda b, i: (b, 0, i)),
        out_shape=jax.ShapeDtypeStruct((_B, 1, _Q), jnp.float32),
        compiler_params=pltpu.CompilerParams(
            dimension_semantics=("parallel", "parallel")),
    )(w, bias, q, k)

    s1, top_idx, v_mean = pl.pallas_call(
        _select_attend_kernel,
        grid=(_B,),
        in_specs=[
            pl.BlockSpec((1, 1, _Q), lambda b: (b, 0, 0)),
            pl.BlockSpec((1, 1, _KV), lambda b: (b, 0, 0)),
            pl.BlockSpec((1, 1, _KV), lambda b: (b, 0, 0)),
            pl.BlockSpec((1, _Q, _D), lambda b: (b, 0, 0)),
            pl.BlockSpec((1, _KV, _D), lambda b: (b, 0, 0)),
            pl.BlockSpec((1, _KV, _D), lambda b: (b, 0, 0)),
        ],
        out_specs=[
            pl.BlockSpec((1, _PAD, _D), lambda b: (b, 0, 0)),
            pl.BlockSpec((1, 1, _PAD), lambda b: (b, 0, 0)),
            pl.BlockSpec((1, 1, _D), lambda b: (b, 0, 0)),
        ],
        out_shape=[
            jax.ShapeDtypeStruct((_B, _PAD, _D), jnp.float32),
            jax.ShapeDtypeStruct((_B, 1, _PAD), jnp.int32),
            jax.ShapeDtypeStruct((_B, 1, _D), jnp.float32),
        ],
        compiler_params=pltpu.CompilerParams(
            dimension_semantics=("parallel",)),
    )(M, w, bias, q, k, v)

    return pl.pallas_call(
        _output_kernel,
        grid=(_B, _NO),
        in_specs=[
            pl.BlockSpec((1, _PAD, _D), lambda b, i: (b, 0, 0)),
            pl.BlockSpec((1, 1, _PAD), lambda b, i: (b, 0, 0)),
            pl.BlockSpec((1, 1, _D), lambda b, i: (b, 0, 0)),
        ],
        out_specs=pl.BlockSpec((1, _OCHUNK, _D), lambda b, i: (b, i, 0)),
        out_shape=jax.ShapeDtypeStruct((_B, _Q, _D), jnp.float32),
        compiler_params=pltpu.CompilerParams(
            dimension_semantics=("parallel", "parallel")),
    )(s1, top_idx, v_mean)


# speedup vs baseline: 3.0184x; 1.0942x over previous
import math

import jax
import jax.numpy as jnp
import numpy as np
from jax.experimental import pallas as pl
from jax.experimental.pallas import tpu as pltpu

_Q = 2048
_KV = 2048
_D = 1024
_B = 2
_C = 5
_U_ACT = int(_C * math.log(_Q))
_U_SAMP = int(_Q * math.log(_KV))
_CAND = 128
_KTH = 64
_BISECT = 34
_PAD = 64
_QCHUNK = 512
_NQ = _Q // _QCHUNK
_OCHUNK = 256
_NO = _Q // _OCHUNK
_NEG_INF = float("-inf")


def _m_kernel(w_ref, bias_ref, q_ref, k_ref, m_ref):
    qb = q_ref[0].astype(jnp.bfloat16)
    kb = k_ref[0].astype(jnp.bfloat16)
    s = jax.lax.dot_general(qb, kb, (((1,), (1,)), ((), ())),
                            preferred_element_type=jnp.float32)
    mx = jnp.max(s + bias_ref[0], axis=1)
    mean = jnp.sum(s * w_ref[0], axis=1)
    m_ref[0, 0, :] = mx - mean


def _select_attend_kernel(m_ref, w_ref, bias_ref, q_ref, k_ref, v_ref,
                          s1_ref, idx_ref, vmean_ref):
    M2 = m_ref[0, 0, :].reshape(16, 128)

    def bis_body(_, carry):
        lo, hi = carry
        mid = 0.5 * (lo + hi)
        cnt = jnp.sum((M2 >= mid).astype(jnp.float32))
        take = cnt >= float(_KTH)
        return jnp.where(take, mid, lo), jnp.where(take, hi, mid)

    t, _ = jax.lax.fori_loop(0, _BISECT, bis_body,
                             (jnp.min(M2), jnp.max(M2)))
    mask = M2 >= t

    lt128 = (jax.lax.broadcasted_iota(jnp.int32, (128, 128), 0)
             < jax.lax.broadcasted_iota(jnp.int32, (128, 128), 1)
             ).astype(jnp.bfloat16)
    lt16 = (jax.lax.broadcasted_iota(jnp.int32, (16, 16), 1)
            < jax.lax.broadcasted_iota(jnp.int32, (16, 16), 0)
            ).astype(jnp.bfloat16)
    mask16 = mask.astype(jnp.bfloat16)
    pin = jax.lax.dot_general(mask16, lt128, (((1,), (0,)), ((), ())),
                              preferred_element_type=jnp.float32)
    rowtot = (pin[:, 127:128]
              + mask16[:, 127:128].astype(jnp.float32)).astype(jnp.bfloat16)
    rowoff = jax.lax.dot_general(lt16, rowtot, (((1,), (0,)), ((), ())),
                                 preferred_element_type=jnp.float32)
    pos = jnp.where(mask, pin + rowoff, -1.0)
    pos_row = jnp.concatenate([pos[r:r + 1, :] for r in range(16)],
                              axis=1).astype(jnp.int32)

    slot_col = jax.lax.broadcasted_iota(jnp.int32, (_CAND, 1), 0)
    ohb = slot_col == pos_row
    onehot = ohb.astype(jnp.float32)
    qiota = jax.lax.broadcasted_iota(jnp.int32, (_CAND, _Q), 1)
    cand_col = jnp.max(jnp.where(ohb, qiota, -1), axis=1,
                       keepdims=True)

    q_cand = jax.lax.dot_general(onehot, q_ref[0], (((1,), (0,)), ((), ())),
                                 preferred_element_type=jnp.float32,
                                 precision=jax.lax.Precision.HIGHEST)
    s_cand = jax.lax.dot_general(q_cand, k_ref[0], (((1,), (1,)), ((), ())),
                                 preferred_element_type=jnp.float32,
                                 precision=jax.lax.Precision.HIGHEST)
    m_cand = (jnp.max(s_cand + bias_ref[0], axis=1, keepdims=True)
              - jnp.sum(s_cand * w_ref[0], axis=1, keepdims=True))
    m_cand = jnp.where(cand_col < 0, _NEG_INF, m_cand)

    slot_row = jax.lax.broadcasted_iota(jnp.int32, (1, _PAD), 1)
    pad_row = jax.lax.broadcasted_iota(jnp.int32, (1, _PAD), 1)

    def sel_body(j, carry):
        mc, lmat_t, fin_row = carry
        mx = jnp.max(mc)
        orig = jnp.min(jnp.where(mc == mx, cand_col, jnp.int32(_Q)))
        slot_mask_col = cand_col == orig
        lmat_t = jnp.where(slot_mask_col & (pad_row == j), 1.0, lmat_t)
        fin_row = jnp.where(slot_row == j, orig, fin_row)
        mc = jnp.where(slot_mask_col, _NEG_INF, mc)
        return mc, lmat_t, fin_row

    lmat_t0 = jnp.zeros((_CAND, _PAD), jnp.float32)
    fin_row0 = jnp.full((1, _PAD), -1, jnp.int32)
    _, lmat_t, fin_row = jax.lax.fori_loop(
        0, _U_ACT, sel_body, (m_cand, lmat_t0, fin_row0))
    idx_ref[0, 0, :] = fin_row[0, :]

    scale = 1.0 / math.sqrt(_KV)
    att = jax.lax.dot_general(lmat_t, s_cand, (((0,), (0,)), ((), ())),
                              preferred_element_type=jnp.float32) * scale
    att = att - jnp.max(att, axis=1, keepdims=True)
    att = jnp.exp(att)
    att = att / jnp.sum(att, axis=1, keepdims=True)
    s1_ref[0] = jax.lax.dot_general(att, v_ref[0], (((1,), (0,)), ((), ())),
                                    preferred_element_type=jnp.float32)
    vmean_ref[0, 0, :] = jnp.mean(v_ref[0], axis=0)


def _output_kernel(s1_ref, idx_ref, vmean_ref, out_ref):
    c = pl.program_id(1)
    onehot_t = ((jax.lax.broadcasted_iota(jnp.int32, (_OCHUNK, _PAD), 0)
                 + c * _OCHUNK) == idx_ref[0]).astype(jnp.float32)
    scattered = jax.lax.dot_general(
        onehot_t, s1_ref[0], (((1,), (0,)), ((), ())),
        preferred_element_type=jnp.float32)
    unsel = 1.0 - jnp.sum(onehot_t, axis=1, keepdims=True)
    out_ref[0] = scattered + unsel * vmean_ref[0]


_IDX = np.asarray(jax.random.randint(jax.random.key(42), (_B, _U_SAMP),
                                     0, _KV))
_COUNTS = np.stack([np.bincount(_IDX[b], minlength=_KV)
                    for b in range(_B)]).astype(np.float32)
_W_HOST = (_COUNTS / np.float32(_U_SAMP)).reshape(_B, 1, _KV)
_BIAS_HOST = np.where(_COUNTS > 0, np.float32(0.0),
                      np.float32(_NEG_INF)).reshape(_B, 1, _KV)


def kernel(q, k, v):
    w = jnp.asarray(_W_HOST)
    bias = jnp.asarray(_BIAS_HOST)

    M = pl.pallas_call(
        _m_kernel,
        grid=(_B, _NQ),
        in_specs=[
            pl.BlockSpec((1, 1, _KV), lambda b, i: (b, 0, 0)),
            pl.BlockSpec((1, 1, _KV), lambda b, i: (b, 0, 0)),
            pl.BlockSpec((1, _QCHUNK, _D), lambda b, i: (b, i, 0)),
            pl.BlockSpec((1, _KV, _D), lambda b, i: (b, 0, 0)),
        ],
        out_specs=pl.BlockSpec((1, 1, _QCHUNK), lambda b, i: (b, 0, i)),
        out_shape=jax.ShapeDtypeStruct((_B, 1, _Q), jnp.float32),
        compiler_params=pltpu.CompilerParams(
            dimension_semantics=("parallel", "parallel")),
    )(w, bias, q, k)

    s1, top_idx, v_mean = pl.pallas_call(
        _select_attend_kernel,
        grid=(_B,),
        in_specs=[
            pl.BlockSpec((1, 1, _Q), lambda b: (b, 0, 0)),
            pl.BlockSpec((1, 1, _KV), lambda b: (b, 0, 0)),
            pl.BlockSpec((1, 1, _KV), lambda b: (b, 0, 0)),
            pl.BlockSpec((1, _Q, _D), lambda b: (b, 0, 0)),
            pl.BlockSpec((1, _KV, _D), lambda b: (b, 0, 0)),
            pl.BlockSpec((1, _KV, _D), lambda b: (b, 0, 0)),
        ],
        out_specs=[
            pl.BlockSpec((1, _PAD, _D), lambda b: (b, 0, 0)),
            pl.BlockSpec((1, 1, _PAD), lambda b: (b, 0, 0)),
            pl.BlockSpec((1, 1, _D), lambda b: (b, 0, 0)),
        ],
        out_shape=[
            jax.ShapeDtypeStruct((_B, _PAD, _D), jnp.float32),
            jax.ShapeDtypeStruct((_B, 1, _PAD), jnp.int32),
            jax.ShapeDtypeStruct((_B, 1, _D), jnp.float32),
        ],
        compiler_params=pltpu.CompilerParams(
            dimension_semantics=("parallel",)),
    )(M, w, bias, q, k, v)

    return pl.pallas_call(
        _output_kernel,
        grid=(_B, _NO),
        in_specs=[
            pl.BlockSpec((1, _PAD, _D), lambda b, i: (b, 0, 0)),
            pl.BlockSpec((1, 1, _PAD), lambda b, i: (b, 0, 0)),
            pl.BlockSpec((1, 1, _D), lambda b, i: (b, 0, 0)),
        ],
        out_specs=pl.BlockSpec((1, _OCHUNK, _D), lambda b, i: (b, i, 0)),
        out_shape=jax.ShapeDtypeStruct((_B, _Q, _D), jnp.float32),
        compiler_params=pltpu.CompilerParams(
            dimension_semantics=("parallel", "parallel")),
    )(s1, top_idx, v_mean)
